# Initial kernel scaffold; baseline (speedup 1.0000x reference)
import jax
import jax.numpy as jnp
from jax.experimental import pallas as pl


def kernel(x, centers):
    raise NotImplementedError("write your pallas kernel here")


# speedup vs baseline: 43.9987x; 43.9987x over previous
import functools

import jax
import jax.numpy as jnp
from jax.experimental import pallas as pl
from jax.experimental.pallas import tpu as pltpu

_BN = 1024


def _body(x_ref, c_ref, o_ref):
    xb = x_ref[...]
    c = c_ref[...]
    s = jax.lax.dot_general(
        xb, c, (((1,), (1,)), ((), ())),
        preferred_element_type=jnp.float32,
    )
    a2 = jnp.sum(xb * xb, axis=1, keepdims=True)
    b2 = jnp.sum(c * c, axis=1)
    d2 = a2 + b2[None, :] - 2.0 * s
    d = jnp.sqrt(jnp.maximum(d2, 0.0))
    m = jnp.min(d, axis=1, keepdims=True)
    K = c.shape[0]
    iota = jax.lax.broadcasted_iota(jnp.int32, d.shape, 1)
    ids = jnp.min(jnp.where(d == m, iota, K), axis=1)
    o_ref[...] = ids.astype(jnp.int32)


@jax.jit
def kernel(x, centers):
    N, D = x.shape
    K = centers.shape[0]
    return pl.pallas_call(
        _body,
        grid=(N // _BN,),
        in_specs=[
            pl.BlockSpec((_BN, D), lambda i: (i, 0)),
            pl.BlockSpec((K, D), lambda i: (0, 0)),
        ],
        out_specs=pl.BlockSpec((_BN,), lambda i: (i,)),
        out_shape=jax.ShapeDtypeStruct((N,), jnp.int32),
    )(x, centers)


# speedup vs baseline: 83.8345x; 1.9054x over previous
import functools

import jax
import jax.numpy as jnp
from jax.experimental import pallas as pl
from jax.experimental.pallas import tpu as pltpu

_BN = 1024


def _body(x_ref, c_ref, b2_ref, o_ref):
    xb = x_ref[...]
    cs = c_ref[...]
    s = jax.lax.dot_general(
        xb, cs, (((1,), (1,)), ((), ())),
        preferred_element_type=jnp.float32,
    )
    d2 = s + b2_ref[...]
    o_ref[...] = jnp.argmin(d2, axis=1).astype(jnp.int32)


@jax.jit
def kernel(x, centers):
    N, D = x.shape
    K = centers.shape[0]
    cs = centers * (-2.0)
    b2 = jnp.sum(centers * centers, axis=1)[None, :]
    return pl.pallas_call(
        _body,
        grid=(N // _BN,),
        in_specs=[
            pl.BlockSpec((_BN, D), lambda i: (i, 0)),
            pl.BlockSpec((K, D), lambda i: (0, 0)),
            pl.BlockSpec((1, K), lambda i: (0, 0)),
        ],
        out_specs=pl.BlockSpec((_BN,), lambda i: (i,)),
        out_shape=jax.ShapeDtypeStruct((N,), jnp.int32),
    )(x, cs, b2)


# speedup vs baseline: 133.8067x; 1.5961x over previous
import functools

import jax
import jax.numpy as jnp
from jax.experimental import pallas as pl
from jax.experimental.pallas import tpu as pltpu

_BN = 1024


def _body(x_ref, c_ref, b2_ref, o_ref):
    xb = x_ref[...]
    cs = c_ref[...]
    s = jax.lax.dot_general(
        cs, xb, (((1,), (1,)), ((), ())),
        preferred_element_type=jnp.float32,
    )
    d2 = s + b2_ref[...]
    o_ref[...] = jnp.argmin(d2, axis=0).astype(jnp.int32)


@jax.jit
def kernel(x, centers):
    N, D = x.shape
    K = centers.shape[0]
    cs = centers * (-2.0)
    b2 = jnp.sum(centers * centers, axis=1)[:, None]
    return pl.pallas_call(
        _body,
        grid=(N // _BN,),
        in_specs=[
            pl.BlockSpec((_BN, D), lambda i: (i, 0)),
            pl.BlockSpec((K, D), lambda i: (0, 0)),
            pl.BlockSpec((K, 1), lambda i: (0, 0)),
        ],
        out_specs=pl.BlockSpec((_BN,), lambda i: (i,)),
        out_shape=jax.ShapeDtypeStruct((N,), jnp.int32),
    )(x, cs, b2)


# speedup vs baseline: 169.7227x; 1.2684x over previous
import functools

import jax
import jax.numpy as jnp
from jax.experimental import pallas as pl
from jax.experimental.pallas import tpu as pltpu

_BN = 4096


def _body(x_ref, c_ref, b2_ref, o_ref):
    xb = x_ref[...]
    cs = c_ref[...]
    s = jax.lax.dot_general(
        cs, xb, (((1,), (1,)), ((), ())),
        preferred_element_type=jnp.float32,
    )
    d2 = s + b2_ref[...]
    o_ref[...] = jnp.argmin(d2, axis=0).astype(jnp.int32)


@jax.jit
def kernel(x, centers):
    N, D = x.shape
    K = centers.shape[0]
    cs = centers * (-2.0)
    b2 = jnp.sum(centers * centers, axis=1)[:, None]
    return pl.pallas_call(
        _body,
        grid=(N // _BN,),
        in_specs=[
            pl.BlockSpec((_BN, D), lambda i: (i, 0)),
            pl.BlockSpec((K, D), lambda i: (0, 0)),
            pl.BlockSpec((K, 1), lambda i: (0, 0)),
        ],
        out_specs=pl.BlockSpec((_BN,), lambda i: (i,)),
        out_shape=jax.ShapeDtypeStruct((N,), jnp.int32),
    )(x, cs, b2)


# speedup vs baseline: 176.6560x; 1.0409x over previous
import functools

import jax
import jax.numpy as jnp
from jax.experimental import pallas as pl
from jax.experimental.pallas import tpu as pltpu

_BN = 4096


def _body(x_ref, c_ref, o_ref):
    xb = x_ref[...]
    xa = jnp.pad(xb, ((0, 0), (0, 1)), constant_values=1.0)
    ca = c_ref[...]
    s = jax.lax.dot_general(
        ca, xa, (((1,), (1,)), ((), ())),
        preferred_element_type=jnp.float32,
    )
    o_ref[...] = jnp.argmin(s, axis=0).astype(jnp.int32)


@jax.jit
def kernel(x, centers):
    N, D = x.shape
    K = centers.shape[0]
    ca = jnp.concatenate(
        [centers * (-2.0), jnp.sum(centers * centers, axis=1)[:, None]], axis=1)
    return pl.pallas_call(
        _body,
        grid=(N // _BN,),
        in_specs=[
            pl.BlockSpec((_BN, D), lambda i: (i, 0)),
            pl.BlockSpec((K, D + 1), lambda i: (0, 0)),
        ],
        out_specs=pl.BlockSpec((_BN,), lambda i: (i,)),
        out_shape=jax.ShapeDtypeStruct((N,), jnp.int32),
    )(x, ca)
